# baseline (speedup 1.0000x reference)
import functools

import jax
import jax.numpy as jnp
from jax import lax
from jax.experimental import pallas as pl
from jax.experimental.pallas import tpu as pltpu
from jax.experimental.pallas import tpu_sc as plsc

N_USER = 25000
N_ITEM = 25000
E = 400000
D_FEAT = 128
CH = 256
OUT = 1
B = 1024

NP = 25088
RB = 256
NSL = 4
SL = CH // NSL


def _enc_body(x_ref, w_ref, b_ref, rt_ref, wt_ref, bt_ref, out_ref):
    x = x_ref[...]
    z = jnp.dot(x, w_ref[...], preferred_element_type=jnp.float32)
    z = jax.nn.relu(z + b_ref[...])
    z = z + rt_ref[...] * wt_ref[...] + bt_ref[...]
    for c in range(NSL):
        out_ref[c, :, :] = z[:, c * SL:(c + 1) * SL]


def _encoder(x, w, b, rt, wt, bt):
    grid = (NP // RB,)
    return pl.pallas_call(
        _enc_body,
        grid=grid,
        in_specs=[
            pl.BlockSpec((RB, D_FEAT), lambda i: (i, 0)),
            pl.BlockSpec((D_FEAT, CH), lambda i: (0, 0)),
            pl.BlockSpec((1, CH), lambda i: (0, 0)),
            pl.BlockSpec((RB, 1), lambda i: (i, 0)),
            pl.BlockSpec((1, CH), lambda i: (0, 0)),
            pl.BlockSpec((1, CH), lambda i: (0, 0)),
        ],
        out_specs=pl.BlockSpec((NSL, RB, SL), lambda i: (0, i, 0)),
        out_shape=jax.ShapeDtypeStruct((NSL, NP, SL), jnp.float32),
    )(x, w, b, rt, wt, bt)


def _layer_body(h_ref, a_ref, rc_ref, ws_ref, wn_ref, b_ref, out_ref):
    rc = rc_ref[...]
    z = b_ref[...]
    for c in range(NSL):
        ksl = pl.ds(c * SL, SL)
        z = z + jnp.dot(h_ref[c, :, :], ws_ref[ksl, :],
                        preferred_element_type=jnp.float32)
        z = z + jnp.dot(a_ref[c, :, :] * rc, wn_ref[ksl, :],
                        preferred_element_type=jnp.float32)
    z = jax.nn.relu(z)
    for c in range(NSL):
        out_ref[c, :, :] = z[:, c * SL:(c + 1) * SL]


def _sage_layer(h4, a4, rcnt, ws, wn, b):
    grid = (NP // RB,)
    return pl.pallas_call(
        _layer_body,
        grid=grid,
        in_specs=[
            pl.BlockSpec((NSL, RB, SL), lambda i: (0, i, 0)),
            pl.BlockSpec((NSL, RB, SL), lambda i: (0, i, 0)),
            pl.BlockSpec((RB, 1), lambda i: (i, 0)),
            pl.BlockSpec((CH, CH), lambda i: (0, 0)),
            pl.BlockSpec((CH, CH), lambda i: (0, 0)),
            pl.BlockSpec((1, CH), lambda i: (0, 0)),
        ],
        out_specs=pl.BlockSpec((NSL, RB, SL), lambda i: (0, i, 0)),
        out_shape=jax.ShapeDtypeStruct((NSL, NP, SL), jnp.float32),
    )(h4, a4, rcnt, ws, wn, b)


def _head_body(h_ref, a_ref, rc_ref, ws_ref, wn_ref, b_ref,
               g_ref, lb_ref, wh_ref, bh_ref, out_ref):
    rc = rc_ref[...]
    z = b_ref[...]
    for c in range(NSL):
        ksl = pl.ds(c * SL, SL)
        z = z + jnp.dot(h_ref[c, :, :], ws_ref[ksl, :],
                        preferred_element_type=jnp.float32)
        z = z + jnp.dot(a_ref[c, :, :] * rc, wn_ref[ksl, :],
                        preferred_element_type=jnp.float32)
    mu = jnp.mean(z, axis=-1, keepdims=True)
    zc = z - mu
    var = jnp.mean(zc * zc, axis=-1, keepdims=True)
    zn = zc * jax.lax.rsqrt(var + 1e-5) * g_ref[...] + lb_ref[...]
    out_ref[...] = jnp.dot(zn, wh_ref[...],
                           preferred_element_type=jnp.float32) + bh_ref[...]


def _head(h4, a4, rcnt, ws, wn, b, g, lb, wh_pad, bh_pad):
    return pl.pallas_call(
        _head_body,
        grid=(1,),
        in_specs=[
            pl.BlockSpec((NSL, B, SL), lambda i: (0, 0, 0)),
            pl.BlockSpec((NSL, B, SL), lambda i: (0, 0, 0)),
            pl.BlockSpec((B, 1), lambda i: (0, 0)),
            pl.BlockSpec((CH, CH), lambda i: (0, 0)),
            pl.BlockSpec((CH, CH), lambda i: (0, 0)),
            pl.BlockSpec((1, CH), lambda i: (0, 0)),
            pl.BlockSpec((1, CH), lambda i: (0, 0)),
            pl.BlockSpec((1, CH), lambda i: (0, 0)),
            pl.BlockSpec((CH, 128), lambda i: (0, 0)),
            pl.BlockSpec((1, 128), lambda i: (0, 0)),
        ],
        out_specs=pl.BlockSpec((B, 128), lambda i: (0, 0)),
        out_shape=jax.ShapeDtypeStruct((B, 128), jnp.float32),
    )(h4, a4, rcnt, ws, wn, b, g, lb, wh_pad, bh_pad)


NC = 2
NS = 16
KB = 128
SLOTS = 2
CR = 40
BPT = 200
EPT = BPT * KB
E_PAD = NS * EPT
RPT = NP // NS
IPC = CR // SLOTS


def _agg_sc_body(h_hbm, src_hbm, dst_hbm, out_hbm, accum, idx_s, idx_d, rows,
                 g0, g1, s0, s1):
    core = lax.axis_index("c")
    sub = lax.axis_index("s")
    r0 = sub * RPT
    gsem = (g0, g1)
    ssem = (s0, s1)

    for csl in range(NSL):
        @pl.when(core == csl // NC)
        def _(csl=csl):
            def zrow(r, carry):
                for c in range(SL // 16):
                    rows[r, pl.ds(c * 16, 16)] = jnp.zeros((16,), jnp.float32)
                return carry
            lax.fori_loop(0, 2 * KB, zrow, 0)

            def zcp(z, carry):
                pltpu.sync_copy(rows.at[pl.ds(0, 2 * KB)],
                                accum.at[pl.ds(r0 + z * 2 * KB, 2 * KB)])
                return carry
            lax.fori_loop(0, RPT // (2 * KB), zcp, 0)
            pltpu.sync_copy(rows.at[pl.ds(0, RPT % (2 * KB))],
                            accum.at[pl.ds(r0 + RPT - RPT % (2 * KB),
                                           RPT % (2 * KB))])
            plsc.subcore_barrier()

            def gather_desc(k, j):
                return pltpu.make_async_copy(
                    h_hbm.at[csl].at[idx_s.at[k]],
                    rows.at[pl.ds(j * KB, KB)], gsem[j])

            def scat_desc(k, j):
                return pltpu.make_async_copy(
                    rows.at[pl.ds(j * KB, KB)],
                    accum.at[idx_d.at[k]], ssem[j])

            def chunk(ci, carry):
                row0 = sub * BPT + ci * CR
                pltpu.sync_copy(src_hbm.at[pl.ds(row0, CR), :], idx_s)
                pltpu.sync_copy(dst_hbm.at[pl.ds(row0, CR), :], idx_d)

                def it(t, c2):
                    for j in range(SLOTS):
                        @pl.when(t > 0)
                        def _(j=j):
                            scat_desc(0, j).wait()
                        gather_desc(t * SLOTS + j, j).start()
                    for j in range(SLOTS):
                        gather_desc(t * SLOTS + j, j).wait()
                        scat_desc(t * SLOTS + j, j).start(add=True)
                    return c2
                lax.fori_loop(0, IPC, it, 0)
                for j in range(SLOTS):
                    scat_desc(0, j).wait()
                return carry
            lax.fori_loop(0, BPT // CR, chunk, 0)

            plsc.subcore_barrier()
            pltpu.sync_copy(accum.at[pl.ds(r0, RPT)],
                            out_hbm.at[csl, pl.ds(r0, RPT)])


def _agg_sc(h4, src2, dst2):
    mesh = plsc.VectorSubcoreMesh(core_axis_name="c", subcore_axis_name="s")
    f = functools.partial(
        pl.kernel, mesh=mesh,
        compiler_params=pltpu.CompilerParams(use_tc_tiling_on_sc=False),
        out_type=jax.ShapeDtypeStruct((NSL, NP, SL), jnp.float32),
        scratch_types=[
            pltpu.VMEM_SHARED((NP, SL), jnp.float32),
            pltpu.VMEM((CR, KB), jnp.int32),
            pltpu.VMEM((CR, KB), jnp.int32),
            pltpu.VMEM((SLOTS * KB, SL), jnp.float32),
        ] + [pltpu.SemaphoreType.DMA] * 4,
    )(_agg_sc_body)
    return f(h4, src2, dst2)


def _pad_edges(edge_index):
    src = jnp.pad(edge_index[0], (0, E_PAD - E))
    dst = jnp.pad(edge_index[1], (0, E_PAD - E), constant_values=NP - 1)
    return src.reshape(E_PAD // KB, KB), dst.reshape(E_PAD // KB, KB)


BP = 1152
SCHUNK = 3200
SROWS = SCHUNK // 16


def _agg_seed_body(h_hbm, src_hbm, dst_hbm, out_hbm,
                   accum, sbuf, dbuf, cs2, cd2, rows, sem):
    core = lax.axis_index("c")
    sub = lax.axis_index("s")
    zp = BP // NS
    iota = lax.iota(jnp.int32, 16)

    for csl in range(NSL):
        @pl.when(core == csl // NC)
        def _(csl=csl):
            def zrow(r, carry):
                for c in range(SL // 16):
                    rows[r, pl.ds(c * 16, 16)] = jnp.zeros((16,), jnp.float32)
                return carry
            lax.fori_loop(0, zp, zrow, 0)
            pltpu.sync_copy(rows.at[pl.ds(0, zp)],
                            accum.at[pl.ds(sub * zp, zp)])
            plsc.subcore_barrier()

            def chunk(ci, carry):
                base = sub * EPT + ci * SCHUNK
                pltpu.sync_copy(src_hbm.at[pl.ds(base, SCHUNK)], sbuf)
                pltpu.sync_copy(dst_hbm.at[pl.ds(base, SCHUNK)], dbuf)

                def scan(i, cnt):
                    s16 = sbuf[pl.ds(i * 16, 16)]
                    d16 = dbuf[pl.ds(i * 16, 16)]
                    mask = d16 < B
                    inc = plsc.cumsum(mask.astype(jnp.int32))
                    pos = cnt + inc - 1
                    plsc.store_scatter(cs2, [pos // KB, pos % KB], s16,
                                       mask=mask)
                    plsc.store_scatter(cd2, [pos // KB, pos % KB], d16,
                                       mask=mask)
                    return cnt + plsc.all_reduce_population_count(mask)
                cnt = lax.fori_loop(0, SROWS, scan, jnp.zeros((16,),
                                                              jnp.int32))

                def pad(p, carry):
                    pos = cnt + p * 16 + iota
                    plsc.store_scatter(cs2, [pos // KB, pos % KB],
                                       jnp.zeros((16,), jnp.int32))
                    plsc.store_scatter(cd2, [pos // KB, pos % KB],
                                       jnp.full((16,), B, jnp.int32))
                    return carry
                lax.fori_loop(0, KB // 16, pad, 0)

                nb = (jnp.max(cnt) + KB - 1) // KB

                def batch(b, carry):
                    pltpu.async_copy(h_hbm.at[csl].at[cs2.at[b]],
                                     rows, sem).wait()
                    pltpu.sync_copy(rows.at[pl.ds(0, KB)],
                                    accum.at[cd2.at[b]], add=True)
                    return carry
                lax.fori_loop(0, nb, batch, 0)
                return carry
            lax.fori_loop(0, EPT // SCHUNK, chunk, 0)

            plsc.subcore_barrier()
            pltpu.sync_copy(accum.at[pl.ds(sub * zp, zp)],
                            out_hbm.at[csl, pl.ds(sub * zp, zp)])


def _agg_sc_seed(h4, src_flat, dst_flat):
    mesh = plsc.VectorSubcoreMesh(core_axis_name="c", subcore_axis_name="s")
    f = functools.partial(
        pl.kernel, mesh=mesh,
        compiler_params=pltpu.CompilerParams(use_tc_tiling_on_sc=False,
                                             needs_layout_passes=False),
        out_type=jax.ShapeDtypeStruct((NSL, BP, SL), jnp.float32),
        scratch_types=[
            pltpu.VMEM_SHARED((BP, SL), jnp.float32),
            pltpu.VMEM((SCHUNK,), jnp.int32),
            pltpu.VMEM((SCHUNK,), jnp.int32),
            pltpu.VMEM((SCHUNK // KB + 1, KB), jnp.int32),
            pltpu.VMEM((SCHUNK // KB + 1, KB), jnp.int32),
            pltpu.VMEM((KB, SL), jnp.float32),
            pltpu.SemaphoreType.DMA,
        ],
    )(_agg_seed_body)
    return f(h4, src_flat, dst_flat)


def _pre_body(dst_iu, dst_ui, seed_time, batch_u, time_u, batch_i, time_i,
              rcnt_u, rcnt_i, rt_u, rt_i,
              accum, ones, zbuf, idx, cbuf, sbuf, bbuf, tbuf, obuf, sem):
    core = lax.axis_index("c")
    sub = lax.axis_index("s")
    r0 = sub * RPT

    def orow(r, carry):
        ones[pl.ds(r * 16, 16)] = jnp.full((16,), 1.0, jnp.float32)
        return carry
    lax.fori_loop(0, KB // 16, orow, 0)

    def zrow(r, carry):
        zbuf[pl.ds(r * 16, 16)] = jnp.zeros((16,), jnp.float32)
        return carry
    lax.fori_loop(0, RPT // 4 // 16, zrow, 0)

    for side in range(2):
        @pl.when(core == side)
        def _(side=side):
            dst2 = dst_iu if side == 0 else dst_ui
            rc_out = rcnt_u if side == 0 else rcnt_i
            rt_out = rt_u if side == 0 else rt_i
            b_in = batch_u if side == 0 else batch_i
            t_in = time_u if side == 0 else time_i

            def zcp(z, carry):
                pltpu.sync_copy(
                    zbuf, accum.at[pl.ds(r0 + z * (RPT // 4), RPT // 4)])
                return carry
            lax.fori_loop(0, 4, zcp, 0)
            plsc.subcore_barrier()

            def chunk(ci, carry):
                row0 = sub * BPT + ci * CR
                pltpu.sync_copy(dst2.at[pl.ds(row0, CR), :], idx)

                def fire(k, c2):
                    pltpu.make_async_copy(
                        ones, accum.at[idx.at[k]], sem).start(add=True)
                    return c2
                lax.fori_loop(0, CR, fire, 0)

                def drain(k, c2):
                    pltpu.make_async_copy(
                        ones, accum.at[idx.at[0]], sem).wait()
                    return c2
                lax.fori_loop(0, CR, drain, 0)
                return carry
            lax.fori_loop(0, BPT // CR, chunk, 0)
            plsc.subcore_barrier()

            pltpu.sync_copy(accum.at[pl.ds(r0, RPT)], cbuf)

            def rrow(i, carry):
                c = cbuf[pl.ds(i * 16, 16)]
                obuf[pl.ds(i * 16, 16)] = 1.0 / jnp.maximum(c, 1.0)
                return carry
            lax.fori_loop(0, RPT // 16, rrow, 0)
            pltpu.sync_copy(obuf, rc_out.at[pl.ds(r0, RPT)])

            pltpu.sync_copy(seed_time, sbuf)
            pltpu.sync_copy(b_in.at[pl.ds(r0, RPT)], bbuf)
            pltpu.sync_copy(t_in.at[pl.ds(r0, RPT)], tbuf)

            def trow(i, carry):
                b16 = bbuf[pl.ds(i * 16, 16)]
                s16 = plsc.load_gather(sbuf, [b16])
                t16 = tbuf[pl.ds(i * 16, 16)]
                obuf[pl.ds(i * 16, 16)] = (
                    (s16 - t16).astype(jnp.float32) * 1e-4)
                return carry
            lax.fori_loop(0, RPT // 16, trow, 0)
            pltpu.sync_copy(obuf, rt_out.at[pl.ds(r0, RPT)])


def _pre_sc(dst_iu, dst_ui, seed_time, batch_u, time_u, batch_i, time_i):
    mesh = plsc.VectorSubcoreMesh(core_axis_name="c", subcore_axis_name="s")
    f = functools.partial(
        pl.kernel, mesh=mesh,
        compiler_params=pltpu.CompilerParams(use_tc_tiling_on_sc=False,
                                             needs_layout_passes=False),
        out_type=[jax.ShapeDtypeStruct((NP,), jnp.float32)] * 4,
        scratch_types=[
            pltpu.VMEM_SHARED((NP,), jnp.float32),
            pltpu.VMEM((KB,), jnp.float32),
            pltpu.VMEM((RPT // 4,), jnp.float32),
            pltpu.VMEM((CR, KB), jnp.int32),
            pltpu.VMEM((RPT,), jnp.float32),
            pltpu.VMEM((1024,), jnp.int32),
            pltpu.VMEM((RPT,), jnp.int32),
            pltpu.VMEM((RPT,), jnp.int32),
            pltpu.VMEM((RPT,), jnp.float32),
            pltpu.SemaphoreType.DMA,
        ],
    )(_pre_body)
    return f(dst_iu, dst_ui, seed_time, batch_u, time_u, batch_i, time_i)


def kernel(x_user, x_item, W_enc_user, b_enc_user, W_enc_item, b_enc_item,
           W_time_user, b_time_user, W_time_item, b_time_item,
           Ws_u1, Wn_u1, b_u1, Ws_i1, Wn_i1, b_i1,
           Ws_u2, Wn_u2, b_u2, Ws_i2, Wn_i2, b_i2,
           ln_g, ln_b, W_head, b_head,
           edge_index_ui, edge_index_iu, seed_time, time_user, time_item,
           batch_user, batch_item):
    xu = jnp.pad(x_user, ((0, NP - N_USER), (0, 0)))
    xi = jnp.pad(x_item, ((0, NP - N_ITEM), (0, 0)))

    src_iu, dst_iu = _pad_edges(edge_index_iu)
    src_ui, dst_ui = _pad_edges(edge_index_ui)

    padn = lambda v: jnp.pad(v, (0, NP - N_USER))
    rcnt_u, rcnt_i, rt_u, rt_i = _pre_sc(
        dst_iu, dst_ui, seed_time,
        padn(batch_user), padn(time_user), padn(batch_item), padn(time_item))
    rcnt_u, rcnt_i = rcnt_u[:, None], rcnt_i[:, None]
    rt_u, rt_i = rt_u[:, None], rt_i[:, None]

    b2 = lambda v: v[None, :]
    hu4 = _encoder(xu, W_enc_user, b2(b_enc_user), rt_u,
                   W_time_user, b2(b_time_user))
    hi4 = _encoder(xi, W_enc_item, b2(b_enc_item), rt_i,
                   W_time_item, b2(b_time_item))

    agg_u1 = _agg_sc(hi4, src_iu, dst_iu)
    agg_i1 = _agg_sc(hu4, src_ui, dst_ui)

    nu4 = _sage_layer(hu4, agg_u1, rcnt_u, Ws_u1, Wn_u1, b2(b_u1))
    ni4 = _sage_layer(hi4, agg_i1, rcnt_i, Ws_i1, Wn_i1, b2(b_i1))

    src_iu_f = jnp.pad(edge_index_iu[0], (0, E_PAD - E))
    dst_iu_f = jnp.pad(edge_index_iu[1], (0, E_PAD - E),
                       constant_values=NP - 1)
    agg_u2 = _agg_sc_seed(ni4, src_iu_f, dst_iu_f)

    wh_pad = jnp.pad(W_head, ((0, 0), (0, 128 - OUT)))
    bh_pad = jnp.pad(b_head, (0, 128 - OUT))[None, :]
    out = _head(nu4, agg_u2[:, :B, :], rcnt_u[:B], Ws_u2, Wn_u2, b2(b_u2),
                b2(ln_g), b2(ln_b), wh_pad, bh_pad)
    return out[:, :OUT]


# speedup vs baseline: 1.1128x; 1.1128x over previous
import functools

import jax
import jax.numpy as jnp
from jax import lax
from jax.experimental import pallas as pl
from jax.experimental.pallas import tpu as pltpu
from jax.experimental.pallas import tpu_sc as plsc

N_USER = 25000
N_ITEM = 25000
E = 400000
D_FEAT = 128
CH = 256
OUT = 1
B = 1024

NP = 25088
RB = 256
NSL = 4
SL = CH // NSL


def _enc_body(x_ref, w_ref, b_ref, rt_ref, wt_ref, bt_ref, out_ref):
    x = x_ref[...]
    z = jnp.dot(x, w_ref[...], preferred_element_type=jnp.float32)
    z = jax.nn.relu(z + b_ref[...])
    z = z + rt_ref[...] * wt_ref[...] + bt_ref[...]
    for c in range(NSL):
        out_ref[c, :, :] = z[:, c * SL:(c + 1) * SL]


def _encoder(x, w, b, rt, wt, bt):
    grid = (NP // RB,)
    return pl.pallas_call(
        _enc_body,
        grid=grid,
        in_specs=[
            pl.BlockSpec((RB, D_FEAT), lambda i: (i, 0)),
            pl.BlockSpec((D_FEAT, CH), lambda i: (0, 0)),
            pl.BlockSpec((1, CH), lambda i: (0, 0)),
            pl.BlockSpec((RB, 1), lambda i: (i, 0)),
            pl.BlockSpec((1, CH), lambda i: (0, 0)),
            pl.BlockSpec((1, CH), lambda i: (0, 0)),
        ],
        out_specs=pl.BlockSpec((NSL, RB, SL), lambda i: (0, i, 0)),
        out_shape=jax.ShapeDtypeStruct((NSL, NP, SL), jnp.float32),
    )(x, w, b, rt, wt, bt)


def _layer_body(h_ref, a_ref, rc_ref, ws_ref, wn_ref, b_ref, out_ref):
    rc = rc_ref[...]
    z = b_ref[...]
    for c in range(NSL):
        ksl = pl.ds(c * SL, SL)
        z = z + jnp.dot(h_ref[c, :, :], ws_ref[ksl, :],
                        preferred_element_type=jnp.float32)
        z = z + jnp.dot(a_ref[c, :, :] * rc, wn_ref[ksl, :],
                        preferred_element_type=jnp.float32)
    z = jax.nn.relu(z)
    for c in range(NSL):
        out_ref[c, :, :] = z[:, c * SL:(c + 1) * SL]


def _sage_layer(h4, a4, rcnt, ws, wn, b):
    grid = (NP // RB,)
    return pl.pallas_call(
        _layer_body,
        grid=grid,
        in_specs=[
            pl.BlockSpec((NSL, RB, SL), lambda i: (0, i, 0)),
            pl.BlockSpec((NSL, RB, SL), lambda i: (0, i, 0)),
            pl.BlockSpec((RB, 1), lambda i: (i, 0)),
            pl.BlockSpec((CH, CH), lambda i: (0, 0)),
            pl.BlockSpec((CH, CH), lambda i: (0, 0)),
            pl.BlockSpec((1, CH), lambda i: (0, 0)),
        ],
        out_specs=pl.BlockSpec((NSL, RB, SL), lambda i: (0, i, 0)),
        out_shape=jax.ShapeDtypeStruct((NSL, NP, SL), jnp.float32),
    )(h4, a4, rcnt, ws, wn, b)


def _head_body(h_ref, a_ref, rc_ref, ws_ref, wn_ref, b_ref,
               g_ref, lb_ref, wh_ref, bh_ref, out_ref):
    rc = rc_ref[...]
    z = b_ref[...]
    for c in range(NSL):
        ksl = pl.ds(c * SL, SL)
        z = z + jnp.dot(h_ref[c, :, :], ws_ref[ksl, :],
                        preferred_element_type=jnp.float32)
        z = z + jnp.dot(a_ref[c, :, :] * rc, wn_ref[ksl, :],
                        preferred_element_type=jnp.float32)
    mu = jnp.mean(z, axis=-1, keepdims=True)
    zc = z - mu
    var = jnp.mean(zc * zc, axis=-1, keepdims=True)
    zn = zc * jax.lax.rsqrt(var + 1e-5) * g_ref[...] + lb_ref[...]
    out_ref[...] = jnp.dot(zn, wh_ref[...],
                           preferred_element_type=jnp.float32) + bh_ref[...]


def _head(h4, a4, rcnt, ws, wn, b, g, lb, wh_pad, bh_pad):
    return pl.pallas_call(
        _head_body,
        grid=(1,),
        in_specs=[
            pl.BlockSpec((NSL, B, SL), lambda i: (0, 0, 0)),
            pl.BlockSpec((NSL, B, SL), lambda i: (0, 0, 0)),
            pl.BlockSpec((B, 1), lambda i: (0, 0)),
            pl.BlockSpec((CH, CH), lambda i: (0, 0)),
            pl.BlockSpec((CH, CH), lambda i: (0, 0)),
            pl.BlockSpec((1, CH), lambda i: (0, 0)),
            pl.BlockSpec((1, CH), lambda i: (0, 0)),
            pl.BlockSpec((1, CH), lambda i: (0, 0)),
            pl.BlockSpec((CH, 128), lambda i: (0, 0)),
            pl.BlockSpec((1, 128), lambda i: (0, 0)),
        ],
        out_specs=pl.BlockSpec((B, 128), lambda i: (0, 0)),
        out_shape=jax.ShapeDtypeStruct((B, 128), jnp.float32),
    )(h4, a4, rcnt, ws, wn, b, g, lb, wh_pad, bh_pad)


NC = 2
NS = 16
KB = 80
SLOTS = 4
CR = 40
BPT = 320
EPT = BPT * KB
E_PAD = NS * EPT
RPT = NP // NS
IPC = CR // SLOTS


def _agg_sc_body(h_hbm, src_hbm, dst_hbm, out_hbm, accum, idx_s, idx_d, rows,
                 g0, g1, g2, g3, s0, s1, s2, s3):
    core = lax.axis_index("c")
    sub = lax.axis_index("s")
    r0 = sub * RPT
    gsem = (g0, g1, g2, g3)
    ssem = (s0, s1, s2, s3)

    for csl in range(NSL):
        @pl.when(core == csl // NC)
        def _(csl=csl):
            def zrow(r, carry):
                for c in range(SL // 16):
                    rows[r, pl.ds(c * 16, 16)] = jnp.zeros((16,), jnp.float32)
                return carry
            lax.fori_loop(0, 2 * KB, zrow, 0)

            def zcp(z, carry):
                pltpu.sync_copy(rows.at[pl.ds(0, 2 * KB)],
                                accum.at[pl.ds(r0 + z * 2 * KB, 2 * KB)])
                return carry
            lax.fori_loop(0, RPT // (2 * KB), zcp, 0)
            pltpu.sync_copy(rows.at[pl.ds(0, RPT % (2 * KB))],
                            accum.at[pl.ds(r0 + RPT - RPT % (2 * KB),
                                           RPT % (2 * KB))])
            plsc.subcore_barrier()

            def gather_desc(k, j):
                return pltpu.make_async_copy(
                    h_hbm.at[csl].at[idx_s.at[k]],
                    rows.at[pl.ds(j * KB, KB)], gsem[j])

            def scat_desc(k, j):
                return pltpu.make_async_copy(
                    rows.at[pl.ds(j * KB, KB)],
                    accum.at[idx_d.at[k]], ssem[j])

            def chunk(ci, carry):
                row0 = sub * BPT + ci * CR
                pltpu.sync_copy(src_hbm.at[pl.ds(row0, CR), :], idx_s)
                pltpu.sync_copy(dst_hbm.at[pl.ds(row0, CR), :], idx_d)

                def it(t, c2):
                    for j in range(SLOTS):
                        @pl.when(t > 0)
                        def _(j=j):
                            scat_desc(0, j).wait()
                        gather_desc(t * SLOTS + j, j).start()
                    for j in range(SLOTS):
                        gather_desc(t * SLOTS + j, j).wait()
                        scat_desc(t * SLOTS + j, j).start(add=True)
                    return c2
                lax.fori_loop(0, IPC, it, 0)
                for j in range(SLOTS):
                    scat_desc(0, j).wait()
                return carry
            lax.fori_loop(0, BPT // CR, chunk, 0)

            plsc.subcore_barrier()
            pltpu.sync_copy(accum.at[pl.ds(r0, RPT)],
                            out_hbm.at[csl, pl.ds(r0, RPT)])


def _agg_sc(h4, src2, dst2):
    mesh = plsc.VectorSubcoreMesh(core_axis_name="c", subcore_axis_name="s")
    f = functools.partial(
        pl.kernel, mesh=mesh,
        compiler_params=pltpu.CompilerParams(use_tc_tiling_on_sc=False),
        out_type=jax.ShapeDtypeStruct((NSL, NP, SL), jnp.float32),
        scratch_types=[
            pltpu.VMEM_SHARED((NP, SL), jnp.float32),
            pltpu.VMEM((CR, KB), jnp.int32),
            pltpu.VMEM((CR, KB), jnp.int32),
            pltpu.VMEM((SLOTS * KB, SL), jnp.float32),
        ] + [pltpu.SemaphoreType.DMA] * 8,
    )(_agg_sc_body)
    return f(h4, src2, dst2)


def _pad_edges(edge_index):
    src = jnp.pad(edge_index[0], (0, E_PAD - E))
    dst = jnp.pad(edge_index[1], (0, E_PAD - E), constant_values=NP - 1)
    return src.reshape(E_PAD // KB, KB), dst.reshape(E_PAD // KB, KB)


BP = 1152
SCHUNK = 3200
SROWS = SCHUNK // 16


def _agg_seed_body(h_hbm, src_hbm, dst_hbm, out_hbm,
                   accum, sbuf, dbuf, cs2, cd2, rows, sem):
    core = lax.axis_index("c")
    sub = lax.axis_index("s")
    zp = BP // NS
    iota = lax.iota(jnp.int32, 16)

    for csl in range(NSL):
        @pl.when(core == csl // NC)
        def _(csl=csl):
            def zrow(r, carry):
                for c in range(SL // 16):
                    rows[r, pl.ds(c * 16, 16)] = jnp.zeros((16,), jnp.float32)
                return carry
            lax.fori_loop(0, zp, zrow, 0)
            pltpu.sync_copy(rows.at[pl.ds(0, zp)],
                            accum.at[pl.ds(sub * zp, zp)])
            plsc.subcore_barrier()

            def chunk(ci, carry):
                base = sub * EPT + ci * SCHUNK
                pltpu.sync_copy(src_hbm.at[pl.ds(base, SCHUNK)], sbuf)
                pltpu.sync_copy(dst_hbm.at[pl.ds(base, SCHUNK)], dbuf)

                def scan(i, cnt):
                    s16 = sbuf[pl.ds(i * 16, 16)]
                    d16 = dbuf[pl.ds(i * 16, 16)]
                    mask = d16 < B
                    inc = plsc.cumsum(mask.astype(jnp.int32))
                    pos = cnt + inc - 1
                    plsc.store_scatter(cs2, [pos // KB, pos % KB], s16,
                                       mask=mask)
                    plsc.store_scatter(cd2, [pos // KB, pos % KB], d16,
                                       mask=mask)
                    return cnt + plsc.all_reduce_population_count(mask)
                cnt = lax.fori_loop(0, SROWS, scan, jnp.zeros((16,),
                                                              jnp.int32))

                def pad(p, carry):
                    pos = cnt + p * 16 + iota
                    plsc.store_scatter(cs2, [pos // KB, pos % KB],
                                       jnp.zeros((16,), jnp.int32))
                    plsc.store_scatter(cd2, [pos // KB, pos % KB],
                                       jnp.full((16,), B, jnp.int32))
                    return carry
                lax.fori_loop(0, KB // 16, pad, 0)

                nb = (jnp.max(cnt) + KB - 1) // KB

                def batch(b, carry):
                    pltpu.async_copy(h_hbm.at[csl].at[cs2.at[b]],
                                     rows, sem).wait()
                    pltpu.sync_copy(rows.at[pl.ds(0, KB)],
                                    accum.at[cd2.at[b]], add=True)
                    return carry
                lax.fori_loop(0, nb, batch, 0)
                return carry
            lax.fori_loop(0, EPT // SCHUNK, chunk, 0)

            plsc.subcore_barrier()
            pltpu.sync_copy(accum.at[pl.ds(sub * zp, zp)],
                            out_hbm.at[csl, pl.ds(sub * zp, zp)])


def _agg_sc_seed(h4, src_flat, dst_flat):
    mesh = plsc.VectorSubcoreMesh(core_axis_name="c", subcore_axis_name="s")
    f = functools.partial(
        pl.kernel, mesh=mesh,
        compiler_params=pltpu.CompilerParams(use_tc_tiling_on_sc=False,
                                             needs_layout_passes=False),
        out_type=jax.ShapeDtypeStruct((NSL, BP, SL), jnp.float32),
        scratch_types=[
            pltpu.VMEM_SHARED((BP, SL), jnp.float32),
            pltpu.VMEM((SCHUNK,), jnp.int32),
            pltpu.VMEM((SCHUNK,), jnp.int32),
            pltpu.VMEM((SCHUNK // KB + 1, KB), jnp.int32),
            pltpu.VMEM((SCHUNK // KB + 1, KB), jnp.int32),
            pltpu.VMEM((KB, SL), jnp.float32),
            pltpu.SemaphoreType.DMA,
        ],
    )(_agg_seed_body)
    return f(h4, src_flat, dst_flat)


def _pre_body(dst_iu, dst_ui, seed_time, batch_u, time_u, batch_i, time_i,
              rcnt_u, rcnt_i, rt_u, rt_i,
              accum, ones, zbuf, idx, cbuf, sbuf, bbuf, tbuf, obuf, sem):
    core = lax.axis_index("c")
    sub = lax.axis_index("s")
    r0 = sub * RPT

    def orow(r, carry):
        ones[pl.ds(r * 16, 16)] = jnp.full((16,), 1.0, jnp.float32)
        return carry
    lax.fori_loop(0, KB // 16, orow, 0)

    def zrow(r, carry):
        zbuf[pl.ds(r * 16, 16)] = jnp.zeros((16,), jnp.float32)
        return carry
    lax.fori_loop(0, RPT // 4 // 16, zrow, 0)

    for side in range(2):
        @pl.when(core == side)
        def _(side=side):
            dst2 = dst_iu if side == 0 else dst_ui
            rc_out = rcnt_u if side == 0 else rcnt_i
            rt_out = rt_u if side == 0 else rt_i
            b_in = batch_u if side == 0 else batch_i
            t_in = time_u if side == 0 else time_i

            def zcp(z, carry):
                pltpu.sync_copy(
                    zbuf, accum.at[pl.ds(r0 + z * (RPT // 4), RPT // 4)])
                return carry
            lax.fori_loop(0, 4, zcp, 0)
            plsc.subcore_barrier()

            def chunk(ci, carry):
                row0 = sub * BPT + ci * CR
                pltpu.sync_copy(dst2.at[pl.ds(row0, CR), :], idx)

                def fire(k, c2):
                    pltpu.make_async_copy(
                        ones, accum.at[idx.at[k]], sem).start(add=True)
                    return c2
                lax.fori_loop(0, CR, fire, 0)

                def drain(k, c2):
                    pltpu.make_async_copy(
                        ones, accum.at[idx.at[0]], sem).wait()
                    return c2
                lax.fori_loop(0, CR, drain, 0)
                return carry
            lax.fori_loop(0, BPT // CR, chunk, 0)
            plsc.subcore_barrier()

            pltpu.sync_copy(accum.at[pl.ds(r0, RPT)], cbuf)

            def rrow(i, carry):
                c = cbuf[pl.ds(i * 16, 16)]
                obuf[pl.ds(i * 16, 16)] = 1.0 / jnp.maximum(c, 1.0)
                return carry
            lax.fori_loop(0, RPT // 16, rrow, 0)
            pltpu.sync_copy(obuf, rc_out.at[pl.ds(r0, RPT)])

            pltpu.sync_copy(seed_time, sbuf)
            pltpu.sync_copy(b_in.at[pl.ds(r0, RPT)], bbuf)
            pltpu.sync_copy(t_in.at[pl.ds(r0, RPT)], tbuf)

            def trow(i, carry):
                b16 = bbuf[pl.ds(i * 16, 16)]
                s16 = plsc.load_gather(sbuf, [b16])
                t16 = tbuf[pl.ds(i * 16, 16)]
                obuf[pl.ds(i * 16, 16)] = (
                    (s16 - t16).astype(jnp.float32) * 1e-4)
                return carry
            lax.fori_loop(0, RPT // 16, trow, 0)
            pltpu.sync_copy(obuf, rt_out.at[pl.ds(r0, RPT)])


def _pre_sc(dst_iu, dst_ui, seed_time, batch_u, time_u, batch_i, time_i):
    mesh = plsc.VectorSubcoreMesh(core_axis_name="c", subcore_axis_name="s")
    f = functools.partial(
        pl.kernel, mesh=mesh,
        compiler_params=pltpu.CompilerParams(use_tc_tiling_on_sc=False,
                                             needs_layout_passes=False),
        out_type=[jax.ShapeDtypeStruct((NP,), jnp.float32)] * 4,
        scratch_types=[
            pltpu.VMEM_SHARED((NP,), jnp.float32),
            pltpu.VMEM((KB,), jnp.float32),
            pltpu.VMEM((RPT // 4,), jnp.float32),
            pltpu.VMEM((CR, KB), jnp.int32),
            pltpu.VMEM((RPT,), jnp.float32),
            pltpu.VMEM((1024,), jnp.int32),
            pltpu.VMEM((RPT,), jnp.int32),
            pltpu.VMEM((RPT,), jnp.int32),
            pltpu.VMEM((RPT,), jnp.float32),
            pltpu.SemaphoreType.DMA,
        ],
    )(_pre_body)
    return f(dst_iu, dst_ui, seed_time, batch_u, time_u, batch_i, time_i)


def kernel(x_user, x_item, W_enc_user, b_enc_user, W_enc_item, b_enc_item,
           W_time_user, b_time_user, W_time_item, b_time_item,
           Ws_u1, Wn_u1, b_u1, Ws_i1, Wn_i1, b_i1,
           Ws_u2, Wn_u2, b_u2, Ws_i2, Wn_i2, b_i2,
           ln_g, ln_b, W_head, b_head,
           edge_index_ui, edge_index_iu, seed_time, time_user, time_item,
           batch_user, batch_item):
    xu = jnp.pad(x_user, ((0, NP - N_USER), (0, 0)))
    xi = jnp.pad(x_item, ((0, NP - N_ITEM), (0, 0)))

    src_iu, dst_iu = _pad_edges(edge_index_iu)
    src_ui, dst_ui = _pad_edges(edge_index_ui)

    padn = lambda v: jnp.pad(v, (0, NP - N_USER))
    rcnt_u, rcnt_i, rt_u, rt_i = _pre_sc(
        dst_iu, dst_ui, seed_time,
        padn(batch_user), padn(time_user), padn(batch_item), padn(time_item))
    rcnt_u, rcnt_i = rcnt_u[:, None], rcnt_i[:, None]
    rt_u, rt_i = rt_u[:, None], rt_i[:, None]

    b2 = lambda v: v[None, :]
    hu4 = _encoder(xu, W_enc_user, b2(b_enc_user), rt_u,
                   W_time_user, b2(b_time_user))
    hi4 = _encoder(xi, W_enc_item, b2(b_enc_item), rt_i,
                   W_time_item, b2(b_time_item))

    agg_u1 = _agg_sc(hi4, src_iu, dst_iu)
    agg_i1 = _agg_sc(hu4, src_ui, dst_ui)

    nu4 = _sage_layer(hu4, agg_u1, rcnt_u, Ws_u1, Wn_u1, b2(b_u1))
    ni4 = _sage_layer(hi4, agg_i1, rcnt_i, Ws_i1, Wn_i1, b2(b_i1))

    src_iu_f = jnp.pad(edge_index_iu[0], (0, E_PAD - E))
    dst_iu_f = jnp.pad(edge_index_iu[1], (0, E_PAD - E),
                       constant_values=NP - 1)
    agg_u2 = _agg_sc_seed(ni4, src_iu_f, dst_iu_f)

    wh_pad = jnp.pad(W_head, ((0, 0), (0, 128 - OUT)))
    bh_pad = jnp.pad(b_head, (0, 128 - OUT))[None, :]
    out = _head(nu4, agg_u2[:, :B, :], rcnt_u[:B], Ws_u2, Wn_u2, b2(b_u2),
                b2(ln_g), b2(ln_b), wh_pad, bh_pad)
    return out[:, :OUT]


# speedup vs baseline: 1.1209x; 1.0073x over previous
import functools

import jax
import jax.numpy as jnp
from jax import lax
from jax.experimental import pallas as pl
from jax.experimental.pallas import tpu as pltpu
from jax.experimental.pallas import tpu_sc as plsc

N_USER = 25000
N_ITEM = 25000
E = 400000
D_FEAT = 128
CH = 256
OUT = 1
B = 1024

NP = 25088
RB = 256
NSL = 4
SL = CH // NSL


def _enc_body(x_ref, w_ref, b_ref, rt_ref, wt_ref, bt_ref, out_ref):
    x = x_ref[...]
    z = jnp.dot(x, w_ref[...], preferred_element_type=jnp.float32)
    z = jax.nn.relu(z + b_ref[...])
    z = z + rt_ref[...] * wt_ref[...] + bt_ref[...]
    for c in range(NSL):
        out_ref[c, :, :] = z[:, c * SL:(c + 1) * SL]


def _encoder(x, w, b, rt, wt, bt):
    grid = (NP // RB,)
    return pl.pallas_call(
        _enc_body,
        grid=grid,
        in_specs=[
            pl.BlockSpec((RB, D_FEAT), lambda i: (i, 0)),
            pl.BlockSpec((D_FEAT, CH), lambda i: (0, 0)),
            pl.BlockSpec((1, CH), lambda i: (0, 0)),
            pl.BlockSpec((RB, 1), lambda i: (i, 0)),
            pl.BlockSpec((1, CH), lambda i: (0, 0)),
            pl.BlockSpec((1, CH), lambda i: (0, 0)),
        ],
        out_specs=pl.BlockSpec((NSL, RB, SL), lambda i: (0, i, 0)),
        out_shape=jax.ShapeDtypeStruct((NSL, NP, SL), jnp.float32),
    )(x, w, b, rt, wt, bt)


def _layer_body(h_ref, a_ref, rc_ref, ws_ref, wn_ref, b_ref, out_ref):
    rc = rc_ref[...]
    z = b_ref[...]
    for c in range(NSL):
        ksl = pl.ds(c * SL, SL)
        z = z + jnp.dot(h_ref[c, :, :], ws_ref[ksl, :],
                        preferred_element_type=jnp.float32)
        z = z + jnp.dot(a_ref[c, :, :] * rc, wn_ref[ksl, :],
                        preferred_element_type=jnp.float32)
    z = jax.nn.relu(z)
    for c in range(NSL):
        out_ref[c, :, :] = z[:, c * SL:(c + 1) * SL]


def _sage_layer(h4, a4, rcnt, ws, wn, b):
    grid = (NP // RB,)
    return pl.pallas_call(
        _layer_body,
        grid=grid,
        in_specs=[
            pl.BlockSpec((NSL, RB, SL), lambda i: (0, i, 0)),
            pl.BlockSpec((NSL, RB, SL), lambda i: (0, i, 0)),
            pl.BlockSpec((RB, 1), lambda i: (i, 0)),
            pl.BlockSpec((CH, CH), lambda i: (0, 0)),
            pl.BlockSpec((CH, CH), lambda i: (0, 0)),
            pl.BlockSpec((1, CH), lambda i: (0, 0)),
        ],
        out_specs=pl.BlockSpec((NSL, RB, SL), lambda i: (0, i, 0)),
        out_shape=jax.ShapeDtypeStruct((NSL, NP, SL), jnp.float32),
    )(h4, a4, rcnt, ws, wn, b)


def _head_body(h_ref, a_ref, rc_ref, ws_ref, wn_ref, b_ref,
               g_ref, lb_ref, wh_ref, bh_ref, out_ref):
    rc = rc_ref[...]
    z = b_ref[...]
    for c in range(NSL):
        ksl = pl.ds(c * SL, SL)
        z = z + jnp.dot(h_ref[c, :, :], ws_ref[ksl, :],
                        preferred_element_type=jnp.float32)
        z = z + jnp.dot(a_ref[c, :, :] * rc, wn_ref[ksl, :],
                        preferred_element_type=jnp.float32)
    mu = jnp.mean(z, axis=-1, keepdims=True)
    zc = z - mu
    var = jnp.mean(zc * zc, axis=-1, keepdims=True)
    zn = zc * jax.lax.rsqrt(var + 1e-5) * g_ref[...] + lb_ref[...]
    out_ref[...] = jnp.dot(zn, wh_ref[...],
                           preferred_element_type=jnp.float32) + bh_ref[...]


def _head(h4, a4, rcnt, ws, wn, b, g, lb, wh_pad, bh_pad):
    return pl.pallas_call(
        _head_body,
        grid=(1,),
        in_specs=[
            pl.BlockSpec((NSL, B, SL), lambda i: (0, 0, 0)),
            pl.BlockSpec((NSL, B, SL), lambda i: (0, 0, 0)),
            pl.BlockSpec((B, 1), lambda i: (0, 0)),
            pl.BlockSpec((CH, CH), lambda i: (0, 0)),
            pl.BlockSpec((CH, CH), lambda i: (0, 0)),
            pl.BlockSpec((1, CH), lambda i: (0, 0)),
            pl.BlockSpec((1, CH), lambda i: (0, 0)),
            pl.BlockSpec((1, CH), lambda i: (0, 0)),
            pl.BlockSpec((CH, 128), lambda i: (0, 0)),
            pl.BlockSpec((1, 128), lambda i: (0, 0)),
        ],
        out_specs=pl.BlockSpec((B, 128), lambda i: (0, 0)),
        out_shape=jax.ShapeDtypeStruct((B, 128), jnp.float32),
    )(h4, a4, rcnt, ws, wn, b, g, lb, wh_pad, bh_pad)


NC = 2
NS = 16
KB = 80
SLOTS = 4
CR = 40
BPT = 320
EPT = BPT * KB
E_PAD = NS * EPT
RPT = NP // NS
IPC = CR // SLOTS


def _agg_sc_body(h_hbm, src_hbm, dst_hbm, out_hbm, accum, idx_s, idx_d, rows,
                 g0, g1, g2, g3, s0, s1, s2, s3):
    core = lax.axis_index("c")
    sub = lax.axis_index("s")
    r0 = sub * RPT
    gsem = (g0, g1, g2, g3)
    ssem = (s0, s1, s2, s3)

    for csl in range(NSL):
        @pl.when(core == csl // NC)
        def _(csl=csl):
            def zrow(r, carry):
                for c in range(SL // 16):
                    rows[r, pl.ds(c * 16, 16)] = jnp.zeros((16,), jnp.float32)
                return carry
            lax.fori_loop(0, 2 * KB, zrow, 0)

            def zcp(z, carry):
                pltpu.sync_copy(rows.at[pl.ds(0, 2 * KB)],
                                accum.at[pl.ds(r0 + z * 2 * KB, 2 * KB)])
                return carry
            lax.fori_loop(0, RPT // (2 * KB), zcp, 0)
            pltpu.sync_copy(rows.at[pl.ds(0, RPT % (2 * KB))],
                            accum.at[pl.ds(r0 + RPT - RPT % (2 * KB),
                                           RPT % (2 * KB))])
            plsc.subcore_barrier()

            def gather_desc(k, j):
                return pltpu.make_async_copy(
                    h_hbm.at[csl].at[idx_s.at[k]],
                    rows.at[pl.ds(j * KB, KB)], gsem[j])

            def scat_desc(k, j):
                return pltpu.make_async_copy(
                    rows.at[pl.ds(j * KB, KB)],
                    accum.at[idx_d.at[k]], ssem[j])

            def chunk(ci, carry):
                row0 = sub * BPT + ci * CR
                pltpu.sync_copy(src_hbm.at[pl.ds(row0, CR), :], idx_s)
                pltpu.sync_copy(dst_hbm.at[pl.ds(row0, CR), :], idx_d)

                def it(t, c2):
                    for j in range(SLOTS):
                        @pl.when(t > 0)
                        def _(j=j):
                            scat_desc(0, j).wait()
                        gather_desc(t * SLOTS + j, j).start()
                    for j in range(SLOTS):
                        gather_desc(t * SLOTS + j, j).wait()
                        scat_desc(t * SLOTS + j, j).start(add=True)
                    return c2
                lax.fori_loop(0, IPC, it, 0)
                for j in range(SLOTS):
                    scat_desc(0, j).wait()
                return carry
            lax.fori_loop(0, BPT // CR, chunk, 0)

            plsc.subcore_barrier()
            pltpu.sync_copy(accum.at[pl.ds(r0, RPT)],
                            out_hbm.at[csl, pl.ds(r0, RPT)])


def _agg_sc(h4, src2, dst2):
    mesh = plsc.VectorSubcoreMesh(core_axis_name="c", subcore_axis_name="s")
    f = functools.partial(
        pl.kernel, mesh=mesh,
        compiler_params=pltpu.CompilerParams(use_tc_tiling_on_sc=False),
        out_type=jax.ShapeDtypeStruct((NSL, NP, SL), jnp.float32),
        scratch_types=[
            pltpu.VMEM_SHARED((NP, SL), jnp.float32),
            pltpu.VMEM((CR, KB), jnp.int32),
            pltpu.VMEM((CR, KB), jnp.int32),
            pltpu.VMEM((SLOTS * KB, SL), jnp.float32),
        ] + [pltpu.SemaphoreType.DMA] * 8,
    )(_agg_sc_body)
    return f(h4, src2, dst2)


def _pad_edges(edge_index):
    src = jnp.pad(edge_index[0], (0, E_PAD - E))
    dst = jnp.pad(edge_index[1], (0, E_PAD - E), constant_values=NP - 1)
    return src.reshape(E_PAD // KB, KB), dst.reshape(E_PAD // KB, KB)


BP = 1152
SCHUNK = 3200
SROWS = SCHUNK // 16
LKB = 64


def _agg_seed_body(h_hbm, src_hbm, dst_hbm, out_hbm,
                   accum, sbuf, dbuf, cs2, cd2, rows, sem):
    core = lax.axis_index("c")
    sub = lax.axis_index("s")
    zp = BP // NS
    iota = lax.iota(jnp.int32, 16)

    for csl in range(NSL):
        @pl.when(core == csl // NC)
        def _(csl=csl):
            def zrow(r, carry):
                for c in range(SL // 16):
                    rows[r, pl.ds(c * 16, 16)] = jnp.zeros((16,), jnp.float32)
                return carry
            lax.fori_loop(0, zp, zrow, 0)
            pltpu.sync_copy(rows.at[pl.ds(0, zp)],
                            accum.at[pl.ds(sub * zp, zp)])
            plsc.subcore_barrier()

            def chunk(ci, carry):
                base = sub * EPT + ci * SCHUNK
                pltpu.sync_copy(src_hbm.at[pl.ds(base, SCHUNK)], sbuf)
                pltpu.sync_copy(dst_hbm.at[pl.ds(base, SCHUNK)], dbuf)

                def scan(i, cnt):
                    s16 = sbuf[pl.ds(i * 16, 16)]
                    d16 = dbuf[pl.ds(i * 16, 16)]
                    mask = d16 < B
                    pc = plsc.all_reduce_population_count(mask)

                    @pl.when(jnp.max(pc) > 0)
                    def _():
                        inc = plsc.cumsum(mask.astype(jnp.int32))
                        pos = cnt + inc - 1
                        plsc.store_scatter(cs2, [pos // LKB, pos % LKB], s16,
                                           mask=mask)
                        plsc.store_scatter(cd2, [pos // LKB, pos % LKB], d16,
                                           mask=mask)
                    return cnt + pc
                cnt = lax.fori_loop(0, SROWS, scan, jnp.zeros((16,),
                                                              jnp.int32))

                def pad(p, carry):
                    pos = cnt + p * 16 + iota
                    plsc.store_scatter(cs2, [pos // LKB, pos % LKB],
                                       jnp.zeros((16,), jnp.int32))
                    plsc.store_scatter(cd2, [pos // LKB, pos % LKB],
                                       jnp.full((16,), B, jnp.int32))
                    return carry
                lax.fori_loop(0, LKB // 16, pad, 0)

                nb = (jnp.max(cnt) + LKB - 1) // LKB

                def batch(b, carry):
                    pltpu.async_copy(h_hbm.at[csl].at[cs2.at[b]],
                                     rows.at[pl.ds(0, LKB)], sem).wait()
                    pltpu.sync_copy(rows.at[pl.ds(0, LKB)],
                                    accum.at[cd2.at[b]], add=True)
                    return carry
                lax.fori_loop(0, nb, batch, 0)
                return carry
            lax.fori_loop(0, EPT // SCHUNK, chunk, 0)

            plsc.subcore_barrier()
            pltpu.sync_copy(accum.at[pl.ds(sub * zp, zp)],
                            out_hbm.at[csl, pl.ds(sub * zp, zp)])


def _agg_sc_seed(h4, src_flat, dst_flat):
    mesh = plsc.VectorSubcoreMesh(core_axis_name="c", subcore_axis_name="s")
    f = functools.partial(
        pl.kernel, mesh=mesh,
        compiler_params=pltpu.CompilerParams(use_tc_tiling_on_sc=False,
                                             needs_layout_passes=False),
        out_type=jax.ShapeDtypeStruct((NSL, BP, SL), jnp.float32),
        scratch_types=[
            pltpu.VMEM_SHARED((BP, SL), jnp.float32),
            pltpu.VMEM((SCHUNK,), jnp.int32),
            pltpu.VMEM((SCHUNK,), jnp.int32),
            pltpu.VMEM((SCHUNK // LKB + 1, LKB), jnp.int32),
            pltpu.VMEM((SCHUNK // LKB + 1, LKB), jnp.int32),
            pltpu.VMEM((KB, SL), jnp.float32),
            pltpu.SemaphoreType.DMA,
        ],
    )(_agg_seed_body)
    return f(h4, src_flat, dst_flat)


def _pre_body(dst_iu, dst_ui, seed_time, batch_u, time_u, batch_i, time_i,
              rcnt_u, rcnt_i, rt_u, rt_i,
              accum, ones, zbuf, idx, cbuf, sbuf, bbuf, tbuf, obuf, sem):
    core = lax.axis_index("c")
    sub = lax.axis_index("s")
    r0 = sub * RPT

    def orow(r, carry):
        ones[pl.ds(r * 16, 16)] = jnp.full((16,), 1.0, jnp.float32)
        return carry
    lax.fori_loop(0, KB // 16, orow, 0)

    def zrow(r, carry):
        zbuf[pl.ds(r * 16, 16)] = jnp.zeros((16,), jnp.float32)
        return carry
    lax.fori_loop(0, RPT // 4 // 16, zrow, 0)

    for side in range(2):
        @pl.when(core == side)
        def _(side=side):
            dst2 = dst_iu if side == 0 else dst_ui
            rc_out = rcnt_u if side == 0 else rcnt_i
            rt_out = rt_u if side == 0 else rt_i
            b_in = batch_u if side == 0 else batch_i
            t_in = time_u if side == 0 else time_i

            def zcp(z, carry):
                pltpu.sync_copy(
                    zbuf, accum.at[pl.ds(r0 + z * (RPT // 4), RPT // 4)])
                return carry
            lax.fori_loop(0, 4, zcp, 0)
            plsc.subcore_barrier()

            def chunk(ci, carry):
                row0 = sub * BPT + ci * CR
                pltpu.sync_copy(dst2.at[pl.ds(row0, CR), :], idx)

                def fire(k, c2):
                    pltpu.make_async_copy(
                        ones, accum.at[idx.at[k]], sem).start(add=True)
                    return c2
                lax.fori_loop(0, CR, fire, 0)

                def drain(k, c2):
                    pltpu.make_async_copy(
                        ones, accum.at[idx.at[0]], sem).wait()
                    return c2
                lax.fori_loop(0, CR, drain, 0)
                return carry
            lax.fori_loop(0, BPT // CR, chunk, 0)
            plsc.subcore_barrier()

            pltpu.sync_copy(accum.at[pl.ds(r0, RPT)], cbuf)

            def rrow(i, carry):
                c = cbuf[pl.ds(i * 16, 16)]
                obuf[pl.ds(i * 16, 16)] = 1.0 / jnp.maximum(c, 1.0)
                return carry
            lax.fori_loop(0, RPT // 16, rrow, 0)
            pltpu.sync_copy(obuf, rc_out.at[pl.ds(r0, RPT)])

            pltpu.sync_copy(seed_time, sbuf)
            pltpu.sync_copy(b_in.at[pl.ds(r0, RPT)], bbuf)
            pltpu.sync_copy(t_in.at[pl.ds(r0, RPT)], tbuf)

            def trow(i, carry):
                b16 = bbuf[pl.ds(i * 16, 16)]
                s16 = plsc.load_gather(sbuf, [b16])
                t16 = tbuf[pl.ds(i * 16, 16)]
                obuf[pl.ds(i * 16, 16)] = (
                    (s16 - t16).astype(jnp.float32) * 1e-4)
                return carry
            lax.fori_loop(0, RPT // 16, trow, 0)
            pltpu.sync_copy(obuf, rt_out.at[pl.ds(r0, RPT)])


def _pre_sc(dst_iu, dst_ui, seed_time, batch_u, time_u, batch_i, time_i):
    mesh = plsc.VectorSubcoreMesh(core_axis_name="c", subcore_axis_name="s")
    f = functools.partial(
        pl.kernel, mesh=mesh,
        compiler_params=pltpu.CompilerParams(use_tc_tiling_on_sc=False,
                                             needs_layout_passes=False),
        out_type=[jax.ShapeDtypeStruct((NP,), jnp.float32)] * 4,
        scratch_types=[
            pltpu.VMEM_SHARED((NP,), jnp.float32),
            pltpu.VMEM((KB,), jnp.float32),
            pltpu.VMEM((RPT // 4,), jnp.float32),
            pltpu.VMEM((CR, KB), jnp.int32),
            pltpu.VMEM((RPT,), jnp.float32),
            pltpu.VMEM((1024,), jnp.int32),
            pltpu.VMEM((RPT,), jnp.int32),
            pltpu.VMEM((RPT,), jnp.int32),
            pltpu.VMEM((RPT,), jnp.float32),
            pltpu.SemaphoreType.DMA,
        ],
    )(_pre_body)
    return f(dst_iu, dst_ui, seed_time, batch_u, time_u, batch_i, time_i)


def kernel(x_user, x_item, W_enc_user, b_enc_user, W_enc_item, b_enc_item,
           W_time_user, b_time_user, W_time_item, b_time_item,
           Ws_u1, Wn_u1, b_u1, Ws_i1, Wn_i1, b_i1,
           Ws_u2, Wn_u2, b_u2, Ws_i2, Wn_i2, b_i2,
           ln_g, ln_b, W_head, b_head,
           edge_index_ui, edge_index_iu, seed_time, time_user, time_item,
           batch_user, batch_item):
    xu = jnp.pad(x_user, ((0, NP - N_USER), (0, 0)))
    xi = jnp.pad(x_item, ((0, NP - N_ITEM), (0, 0)))

    src_iu, dst_iu = _pad_edges(edge_index_iu)
    src_ui, dst_ui = _pad_edges(edge_index_ui)

    padn = lambda v: jnp.pad(v, (0, NP - N_USER))
    rcnt_u, rcnt_i, rt_u, rt_i = _pre_sc(
        dst_iu, dst_ui, seed_time,
        padn(batch_user), padn(time_user), padn(batch_item), padn(time_item))
    rcnt_u, rcnt_i = rcnt_u[:, None], rcnt_i[:, None]
    rt_u, rt_i = rt_u[:, None], rt_i[:, None]

    b2 = lambda v: v[None, :]
    hu4 = _encoder(xu, W_enc_user, b2(b_enc_user), rt_u,
                   W_time_user, b2(b_time_user))
    hi4 = _encoder(xi, W_enc_item, b2(b_enc_item), rt_i,
                   W_time_item, b2(b_time_item))

    agg_u1 = _agg_sc(hi4, src_iu, dst_iu)
    agg_i1 = _agg_sc(hu4, src_ui, dst_ui)

    nu4 = _sage_layer(hu4, agg_u1, rcnt_u, Ws_u1, Wn_u1, b2(b_u1))
    ni4 = _sage_layer(hi4, agg_i1, rcnt_i, Ws_i1, Wn_i1, b2(b_i1))

    src_iu_f = jnp.pad(edge_index_iu[0], (0, E_PAD - E))
    dst_iu_f = jnp.pad(edge_index_iu[1], (0, E_PAD - E),
                       constant_values=NP - 1)
    agg_u2 = _agg_sc_seed(ni4, src_iu_f, dst_iu_f)

    wh_pad = jnp.pad(W_head, ((0, 0), (0, 128 - OUT)))
    bh_pad = jnp.pad(b_head, (0, 128 - OUT))[None, :]
    out = _head(nu4, agg_u2[:, :B, :], rcnt_u[:B], Ws_u2, Wn_u2, b2(b_u2),
                b2(ln_g), b2(ln_b), wh_pad, bh_pad)
    return out[:, :OUT]
